# baseline (speedup 1.0000x reference)
import functools

import jax
import jax.numpy as jnp
from jax import lax
from jax.experimental import pallas as pl
from jax.experimental.pallas import tpu as pltpu
from jax.experimental.pallas import tpu_sc as plsc


def _sc_broadcast_rows(pe2d, batch, length):
    d = pe2d.shape[1]
    info = plsc.get_sparse_core_info()
    nc, ns = info.num_cores, info.num_subcores
    nw = nc * ns
    slab = 32
    rows_per_w = -(-length // nw)
    rows_per_w = -(-rows_per_w // slab) * slab
    nslab = rows_per_w // slab
    assert length >= rows_per_w

    mesh = plsc.VectorSubcoreMesh(core_axis_name="c", subcore_axis_name="s")

    @functools.partial(
        pl.kernel,
        mesh=mesh,
        out_type=jax.ShapeDtypeStruct((batch, length, d), jnp.float32),
        scratch_types=[pltpu.VMEM((slab, d), jnp.float32)],
        compiler_params=pltpu.CompilerParams(use_tc_tiling_on_sc=False),
    )
    def k(pe_hbm, out_hbm, buf):
        wid = lax.axis_index("s") * nc + lax.axis_index("c")
        base = jnp.minimum(wid * rows_per_w, length - rows_per_w)
        for si in range(nslab):
            r0 = base + si * slab
            pltpu.sync_copy(pe_hbm.at[pl.ds(r0, slab), :], buf)
            for b in range(batch):
                pltpu.sync_copy(buf, out_hbm.at[b, pl.ds(r0, slab), :])

    return k(pe2d)


def kernel(x, pe):
    b, s, _ = x.shape
    length = 2 * s - 1
    return _sc_broadcast_rows(pe[0], b, length)


# speedup vs baseline: 1.0102x; 1.0102x over previous
import functools

import jax
import jax.numpy as jnp
from jax import lax
from jax.experimental import pallas as pl
from jax.experimental.pallas import tpu as pltpu
from jax.experimental.pallas import tpu_sc as plsc


def _sc_broadcast_rows(pe2d, batch, length):
    d = pe2d.shape[1]
    info = plsc.get_sparse_core_info()
    nc, ns = info.num_cores, info.num_subcores
    nw = nc * ns
    slab = 32
    rows_per_w = -(-length // nw)
    rows_per_w = -(-rows_per_w // slab) * slab
    nslab = rows_per_w // slab
    assert length >= rows_per_w

    mesh = plsc.VectorSubcoreMesh(core_axis_name="c", subcore_axis_name="s")

    nbuf = 3

    @functools.partial(
        pl.kernel,
        mesh=mesh,
        out_type=jax.ShapeDtypeStruct((batch, length, d), jnp.float32),
        scratch_types=[
            [pltpu.VMEM((slab, d), jnp.float32) for _ in range(nbuf)],
            [pltpu.SemaphoreType.DMA for _ in range(nbuf)],
            [pltpu.SemaphoreType.DMA for _ in range(nbuf)],
        ],
        compiler_params=pltpu.CompilerParams(use_tc_tiling_on_sc=False),
    )
    def k(pe_hbm, out_hbm, bufs, gsems, wsems):
        wid = lax.axis_index("s") * nc + lax.axis_index("c")
        base = jnp.minimum(wid * rows_per_w, length - rows_per_w)

        def gcopy(i):
            return pltpu.make_async_copy(
                pe_hbm.at[pl.ds(base + i * slab, slab), :],
                bufs[i % nbuf], gsems[i % nbuf])

        def wcopy(i, b):
            return pltpu.make_async_copy(
                bufs[i % nbuf],
                out_hbm.at[b, pl.ds(base + i * slab, slab), :],
                wsems[i % nbuf])

        gcopy(0).start()
        if nslab > 1:
            gcopy(1).start()
        for i in range(nslab):
            gcopy(i).wait()
            for b in range(batch):
                wcopy(i, b).start()
            if i > 0:
                for b in range(batch):
                    wcopy(i - 1, b).wait()
            if i + 2 < nslab:
                gcopy(i + 2).start()
        for b in range(batch):
            wcopy(nslab - 1, b).wait()

    return k(pe2d)


def kernel(x, pe):
    b, s, _ = x.shape
    length = 2 * s - 1
    return _sc_broadcast_rows(pe[0], b, length)
